# Initial kernel scaffold; baseline (speedup 1.0000x reference)
import jax
import jax.numpy as jnp
from jax.experimental import pallas as pl


def kernel(user_ids, item_ids, user_table, item_table, u_W1, u_b1, u_W2, u_b2, u_W3, u_b3, i_W1, i_b1, i_W2, i_b2, i_W3, i_b3, p_W1, p_b1, p_W2, p_b2):
    raise NotImplementedError("write your pallas kernel here")


# speedup vs baseline: 1.1899x; 1.1899x over previous
import jax
import jax.numpy as jnp
from jax.experimental import pallas as pl
from jax.experimental.pallas import tpu as pltpu
from jax.experimental.pallas import tpu_sc as plsc

GATHER_WINDOW = 128


def _sc_gather_both(user_ids, item_ids, user_table, item_table):
    B = user_ids.shape[0]
    H = user_table.shape[1]
    ui = user_ids.reshape(1, B)
    ii = item_ids.reshape(1, B)
    mesh = plsc.VectorSubcoreMesh(core_axis_name="core", subcore_axis_name="subcore")

    @pl.kernel(
        out_type=(
            jax.ShapeDtypeStruct((B, H), user_table.dtype),
            jax.ShapeDtypeStruct((B, H), item_table.dtype),
        ),
        mesh=mesh,
    )
    def gather_kernel(ut_hbm, it_hbm, ui_hbm, ii_hbm, uo_hbm, io_hbm):
        def body(ui_vmem, ii_vmem, uo_vmem, io_vmem):
            pltpu.sync_copy(ut_hbm.at[ui_vmem.at[0]], uo_vmem)
            pltpu.sync_copy(it_hbm.at[ii_vmem.at[0]], io_vmem)

        pltpu.emit_pipeline(
            body,
            grid=(B // GATHER_WINDOW,),
            in_specs=[
                pl.BlockSpec((1, GATHER_WINDOW), lambda i: (0, i)),
                pl.BlockSpec((1, GATHER_WINDOW), lambda i: (0, i)),
            ],
            out_specs=[
                pl.BlockSpec((GATHER_WINDOW, H), lambda i: (i, 0)),
                pl.BlockSpec((GATHER_WINDOW, H), lambda i: (i, 0)),
            ],
            core_axis_name=("core", "subcore"),
            dimension_semantics=(pltpu.PARALLEL,),
        )(ui_hbm, ii_hbm, uo_hbm, io_hbm)

    return gather_kernel(user_table, item_table, ui, ii)


def _mlp_body(ue_ref, ie_ref,
              u_W1, u_b1, u_W2, u_b2, u_W3, u_b3,
              i_W1, i_b1, i_W2, i_b2, i_W3, i_b3,
              p_W1, p_b1, p_W2, p_b2, out_ref):
    f32 = jnp.float32
    ue = ue_ref[...]
    ue = jnp.maximum(jnp.dot(ue, u_W1[...], preferred_element_type=f32) + u_b1[...], 0.0)
    ue = jnp.maximum(jnp.dot(ue, u_W2[...], preferred_element_type=f32) + u_b2[...], 0.0)
    ue = jnp.maximum(jnp.dot(ue, u_W3[...], preferred_element_type=f32) + u_b3[...], 0.0)
    ie = ie_ref[...]
    ie = jnp.maximum(jnp.dot(ie, i_W1[...], preferred_element_type=f32) + i_b1[...], 0.0)
    ie = jnp.maximum(jnp.dot(ie, i_W2[...], preferred_element_type=f32) + i_b2[...], 0.0)
    ie = jnp.maximum(jnp.dot(ie, i_W3[...], preferred_element_type=f32) + i_b3[...], 0.0)
    H = ue.shape[1]
    h = (jnp.dot(ue, p_W1[:H, :], preferred_element_type=f32)
         + jnp.dot(ie, p_W1[H:, :], preferred_element_type=f32) + p_b1[...])
    h = jnp.maximum(h, 0.0)
    out_ref[...] = jnp.dot(h, p_W2[...], preferred_element_type=f32) + p_b2[...]


def kernel(user_ids, item_ids, user_table, item_table,
           u_W1, u_b1, u_W2, u_b2, u_W3, u_b3,
           i_W1, i_b1, i_W2, i_b2, i_W3, i_b3,
           p_W1, p_b1, p_W2, p_b2):
    B = user_ids.shape[0]
    H = user_table.shape[1]
    ue = jnp.take(user_table, user_ids, axis=0)
    ie = jnp.take(item_table, item_ids, axis=0)

    BLK = 2048
    full = lambda shape: pl.BlockSpec(shape, lambda i: tuple(0 for _ in shape))
    preds = pl.pallas_call(
        _mlp_body,
        grid=(B // BLK,),
        in_specs=[
            pl.BlockSpec((BLK, H), lambda i: (i, 0)),
            pl.BlockSpec((BLK, H), lambda i: (i, 0)),
            full(u_W1.shape), full(u_b1.shape), full(u_W2.shape), full(u_b2.shape),
            full(u_W3.shape), full(u_b3.shape),
            full(i_W1.shape), full(i_b1.shape), full(i_W2.shape), full(i_b2.shape),
            full(i_W3.shape), full(i_b3.shape),
            full(p_W1.shape), full(p_b1.shape), full(p_W2.shape), full(p_b2.shape),
        ],
        out_specs=pl.BlockSpec((BLK, 1), lambda i: (i, 0)),
        out_shape=jax.ShapeDtypeStruct((B, 1), jnp.float32),
    )(ue, ie,
      u_W1, u_b1, u_W2, u_b2, u_W3, u_b3,
      i_W1, i_b1, i_W2, i_b2, i_W3, i_b3,
      p_W1, p_b1, p_W2, p_b2)
    return preds.reshape(-1)
